# Initial kernel scaffold; baseline (speedup 1.0000x reference)
import jax
import jax.numpy as jnp
from jax.experimental import pallas as pl


def kernel(x, edge_index, params):
    raise NotImplementedError("write your pallas kernel here")


# speedup vs baseline: 18.2943x; 18.2943x over previous
import functools

import jax
import jax.numpy as jnp
from jax import lax
from jax.experimental import pallas as pl
from jax.experimental.pallas import tpu as pltpu
from jax.experimental.pallas import tpu_sc as plsc

N = 10000
E = 640000
D = 128
H = 32
C = 16
EPS_BN = 1e-5
EPS_LN = 1e-5

NC = 2
NS = 16
NW = NC * NS
B = 128
NP = 10016
CHUNKS = -(-E // (NW * B))
EPW = CHUNKS * B
EPAD = NW * EPW


def _sc_degree(dst_hbm, ones_hbm, zeros_hbm, out_hbm, idx_v, ones_v, deg_sh):
    c = lax.axis_index("c")
    s = lax.axis_index("s")
    wid = c * NS + s

    @pl.when(s == 0)
    def _():
        pltpu.sync_copy(zeros_hbm, deg_sh)

    pltpu.sync_copy(ones_hbm, ones_v)
    plsc.subcore_barrier()

    def body(i, carry):
        base = wid * EPW + i * B
        pltpu.sync_copy(dst_hbm.at[pl.ds(base, B)], idx_v)
        pltpu.sync_copy(ones_v, deg_sh.at[idx_v], add=True)
        return carry

    lax.fori_loop(0, CHUNKS, body, 0)
    plsc.subcore_barrier()

    @pl.when(s == 0)
    def _():
        pltpu.sync_copy(deg_sh, out_hbm.at[c])


def _sc_scatter(hs_hbm, src_hbm, dst_hbm, zeros_hbm, out_hbm,
                sidx, didx, rows, agg_sh, sem):
    c = lax.axis_index("c")
    s = lax.axis_index("s")
    wid = c * NS + s

    @pl.when(s == 0)
    def _():
        pltpu.sync_copy(zeros_hbm, agg_sh)

    plsc.subcore_barrier()

    def body(i, carry):
        base = wid * EPW + i * B
        pltpu.sync_copy(src_hbm.at[pl.ds(base, B)], sidx)
        pltpu.async_copy(hs_hbm.at[sidx], rows, sem).wait()
        pltpu.sync_copy(dst_hbm.at[pl.ds(base, B)], didx)
        pltpu.sync_copy(rows, agg_sh.at[didx], add=True)
        return carry

    lax.fori_loop(0, CHUNKS, body, 0)
    plsc.subcore_barrier()

    @pl.when(s == 0)
    def _():
        pltpu.sync_copy(agg_sh, out_hbm.at[c])


@functools.lru_cache(maxsize=None)
def _sc_kernels():
    mesh = plsc.VectorSubcoreMesh(
        core_axis_name="c", subcore_axis_name="s", num_cores=NC, num_subcores=NS
    )
    cp = pltpu.CompilerParams(use_tc_tiling_on_sc=False)
    degree = pl.kernel(
        _sc_degree,
        out_type=jax.ShapeDtypeStruct((NC, NP, 8), jnp.float32),
        mesh=mesh,
        compiler_params=cp,
        scratch_types=[
            pltpu.VMEM((B,), jnp.int32),
            pltpu.VMEM((B, 8), jnp.float32),
            pltpu.VMEM_SHARED((NP, 8), jnp.float32),
        ],
    )
    scatter = pl.kernel(
        _sc_scatter,
        out_type=jax.ShapeDtypeStruct((NC, NP, H), jnp.float32),
        mesh=mesh,
        compiler_params=cp,
        scratch_types=[
            pltpu.VMEM((B,), jnp.int32),
            pltpu.VMEM((B,), jnp.int32),
            pltpu.VMEM((B, H), jnp.float32),
            pltpu.VMEM_SHARED((NP, H), jnp.float32),
            pltpu.SemaphoreType.DMA,
        ],
    )
    return degree, scatter


def _tc_first(deg_ref, x_ref, w_ref, dinv_ref, hs_ref):
    deg = deg_ref[0] + deg_ref[1]
    dinv = lax.rsqrt(deg + 1.0)
    dinv_ref[...] = dinv
    h = jnp.dot(x_ref[...], w_ref[...], preferred_element_type=jnp.float32)
    hs_ref[...] = h * dinv[:, 0:1]


def _tc_mid(agg_ref, hs_ref, dinv_ref, pb_ref, w_ref, out_ref):
    dinv = dinv_ref[...][:, 0:1]
    t = dinv * (agg_ref[0] + agg_ref[1] + hs_ref[...]) + pb_ref[0:1, :]
    tr = t[:N]
    mu = jnp.mean(tr, axis=0, keepdims=True)
    var = jnp.mean((tr - mu) ** 2, axis=0, keepdims=True)
    hn = (t - mu) * lax.rsqrt(var + EPS_BN) * pb_ref[1:2, :] + pb_ref[2:3, :]
    h = jnp.maximum(hn, 0.0)
    hs2 = jnp.dot(h, w_ref[...], preferred_element_type=jnp.float32) * dinv
    rowid = lax.broadcasted_iota(jnp.int32, (NP, H), 0)
    out_ref[...] = jnp.where(rowid < N, hs2, 0.0)


def _tc_final(agg_ref, hs_ref, dinv_ref, pm_ref,
              w1_ref, w2_ref, w3_ref, w4_ref, wo_ref, out_ref):
    dinv = dinv_ref[...][:, 0:1]
    t = dinv * (agg_ref[0] + agg_ref[1] + hs_ref[...]) + pm_ref[0:1, :]
    tr = t[:N]
    mu = jnp.mean(tr, axis=0, keepdims=True)
    var = jnp.mean((tr - mu) ** 2, axis=0, keepdims=True)
    h = jnp.maximum(
        (tr - mu) * lax.rsqrt(var + EPS_BN) * pm_ref[1:2, :] + pm_ref[2:3, :],
        0.0,
    )
    ws = [w1_ref, w2_ref, w3_ref, w4_ref]
    for j in range(4):
        h = jnp.dot(h, ws[j][...], preferred_element_type=jnp.float32)
        h = jnp.maximum(h + pm_ref[3 + 3 * j:4 + 3 * j, :], 0.0)
        mu2 = jnp.mean(h, axis=-1, keepdims=True)
        var2 = jnp.mean((h - mu2) ** 2, axis=-1, keepdims=True)
        h = ((h - mu2) * lax.rsqrt(var2 + EPS_LN) * pm_ref[4 + 3 * j:5 + 3 * j, :]
             + pm_ref[5 + 3 * j:6 + 3 * j, :])
    out = jnp.dot(h, wo_ref[...], preferred_element_type=jnp.float32)
    out_ref[...] = out + pm_ref[15:16, :C]


def _sd(shape):
    return jax.ShapeDtypeStruct(shape, jnp.float32)


def kernel(x, edge_index, params):
    src = edge_index[0]
    dst = edge_index[1]
    pad = jnp.full((EPAD - E,), N, jnp.int32)
    src_p = jnp.concatenate([src, pad])
    dst_p = jnp.concatenate([dst, pad])
    x_p = jnp.zeros((NP, D), jnp.float32).at[:N].set(x)
    zeros8 = jnp.zeros((NP, 8), jnp.float32)
    zerosH = jnp.zeros((NP, H), jnp.float32)
    ones8 = jnp.ones((B, 8), jnp.float32)

    sc_degree, sc_scatter = _sc_kernels()
    deg_part = sc_degree(dst_p, ones8, zeros8)
    dinv, hs = pl.pallas_call(
        _tc_first, out_shape=[_sd((NP, 8)), _sd((NP, H))]
    )(deg_part, x_p, params['conv_W'][0])

    out = None
    for i in range(3):
        agg = sc_scatter(hs, src_p, dst_p, zerosH)
        if i < 2:
            pb = jnp.concatenate([
                params['conv_b'][i][None, :],
                params['bn_g'][i][None, :],
                params['bn_b'][i][None, :],
                jnp.zeros((5, H), jnp.float32),
            ])
            hs = pl.pallas_call(_tc_mid, out_shape=_sd((NP, H)))(
                agg, hs, dinv, pb, params['conv_W'][i + 1]
            )
        else:
            rows = [params['conv_b'][2][None, :],
                    params['bn_g'][2][None, :],
                    params['bn_b'][2][None, :]]
            for j in range(4):
                rows += [params['head_b'][j][None, :],
                         params['ln_g'][j][None, :],
                         params['ln_b'][j][None, :]]
            rows.append(jnp.pad(params['out_b'], (0, H - C))[None, :])
            pm = jnp.concatenate(rows)
            out = pl.pallas_call(_tc_final, out_shape=_sd((N, C)))(
                agg, hs, dinv, pm,
                params['head_W'][0], params['head_W'][1],
                params['head_W'][2], params['head_W'][3],
                params['out_W'],
            )
    return out


# speedup vs baseline: 25.6875x; 1.4041x over previous
import functools

import jax
import jax.numpy as jnp
from jax import lax
from jax.experimental import pallas as pl
from jax.experimental.pallas import tpu as pltpu
from jax.experimental.pallas import tpu_sc as plsc

N = 10000
E = 640000
D = 128
H = 32
C = 16
EPS_BN = 1e-5
EPS_LN = 1e-5

NC = 2
NS = 16
NW = NC * NS
B = 128
NP = 10016
RING = 4
CHUNKS = 160
EPW = CHUNKS * B
EPAD = NW * EPW
RPT = NP // NS


def _sc_degree(dst_hbm, ones_hbm, zeros_hbm, out_hbm,
               didx_all, ones_v, deg_sh, ssem):
    c = lax.axis_index("c")
    s = lax.axis_index("s")
    wid = c * NS + s

    pltpu.sync_copy(zeros_hbm.at[pl.ds(s * RPT, RPT)],
                    deg_sh.at[pl.ds(s * RPT, RPT)])
    pltpu.sync_copy(dst_hbm.at[wid], didx_all)
    pltpu.sync_copy(ones_hbm, ones_v)
    plsc.subcore_barrier()

    K = 8

    def body(i, carry):
        for b in range(K):
            pltpu.async_copy(ones_v, deg_sh.at[didx_all.at[K * i + b]],
                             ssem, add=True)
        for b in range(K):
            pltpu.make_async_copy(
                ones_v, deg_sh.at[didx_all.at[K * i + b]], ssem).wait()
        return carry

    lax.fori_loop(0, CHUNKS // K, body, 0)
    plsc.subcore_barrier()
    pltpu.sync_copy(deg_sh.at[pl.ds(s * RPT, RPT)],
                    out_hbm.at[c, pl.ds(s * RPT, RPT)])


def _sc_scatter(hs_hbm, src_hbm, dst_hbm, zeros_hbm, out_hbm,
                sidx_all, didx_all, rows0, rows1, rows2, rows3,
                agg_sh, g0, g1, g2, g3, s0, s1, s2, s3):
    c = lax.axis_index("c")
    s = lax.axis_index("s")
    wid = c * NS + s
    rows = [rows0, rows1, rows2, rows3]
    gsem = [g0, g1, g2, g3]
    ssem = [s0, s1, s2, s3]

    pltpu.sync_copy(zeros_hbm.at[pl.ds(s * RPT, RPT)],
                    agg_sh.at[pl.ds(s * RPT, RPT)])
    pltpu.sync_copy(src_hbm.at[wid], sidx_all)
    pltpu.sync_copy(dst_hbm.at[wid], didx_all)
    plsc.subcore_barrier()

    for b in range(RING):
        pltpu.make_async_copy(hs_hbm.at[sidx_all.at[b]],
                              rows[b], gsem[b]).start()

    def body(i, carry):
        for b in range(RING):
            pltpu.make_async_copy(hs_hbm.at[sidx_all.at[RING * i + b]],
                                  rows[b], gsem[b]).wait()
        for b in range(RING):
            pltpu.async_copy(rows[b], agg_sh.at[didx_all.at[RING * i + b]],
                             ssem[b], add=True)
        for b in range(RING):
            pltpu.make_async_copy(
                rows[b], agg_sh.at[didx_all.at[RING * i + b]], ssem[b]).wait()

        @pl.when(i < CHUNKS // RING - 1)
        def _():
            for b in range(RING):
                pltpu.make_async_copy(
                    hs_hbm.at[sidx_all.at[RING * (i + 1) + b]],
                    rows[b], gsem[b]).start()
        return carry

    lax.fori_loop(0, CHUNKS // RING, body, 0)
    plsc.subcore_barrier()
    pltpu.sync_copy(agg_sh.at[pl.ds(s * RPT, RPT)],
                    out_hbm.at[c, pl.ds(s * RPT, RPT)])


@functools.lru_cache(maxsize=None)
def _sc_kernels():
    mesh = plsc.VectorSubcoreMesh(
        core_axis_name="c", subcore_axis_name="s", num_cores=NC, num_subcores=NS
    )
    cp = pltpu.CompilerParams(use_tc_tiling_on_sc=False)
    degree = pl.kernel(
        _sc_degree,
        out_type=jax.ShapeDtypeStruct((NC, NP, 8), jnp.float32),
        mesh=mesh,
        compiler_params=cp,
        scratch_types=[
            pltpu.VMEM((CHUNKS, B), jnp.int32),
            pltpu.VMEM((B, 8), jnp.float32),
            pltpu.VMEM_SHARED((NP, 8), jnp.float32),
            pltpu.SemaphoreType.DMA,
        ],
    )
    scatter = pl.kernel(
        _sc_scatter,
        out_type=jax.ShapeDtypeStruct((NC, NP, H), jnp.float32),
        mesh=mesh,
        compiler_params=cp,
        scratch_types=(
            [pltpu.VMEM((CHUNKS, B), jnp.int32)] * 2
            + [pltpu.VMEM((B, H), jnp.float32)] * RING
            + [pltpu.VMEM_SHARED((NP, H), jnp.float32)]
            + [pltpu.SemaphoreType.DMA] * (2 * RING)
        ),
    )
    return degree, scatter


def _tc_first(deg_ref, x_ref, w_ref, dinv_ref, hs_ref):
    deg = deg_ref[0] + deg_ref[1]
    dinv = lax.rsqrt(deg + 1.0)
    dinv_ref[...] = dinv
    h = jnp.dot(x_ref[...], w_ref[...], preferred_element_type=jnp.float32)
    hs_ref[...] = h * dinv[:, 0:1]


def _tc_mid(agg_ref, hs_ref, dinv_ref, pb_ref, w_ref, out_ref):
    dinv = dinv_ref[...][:, 0:1]
    t = dinv * (agg_ref[0] + agg_ref[1] + hs_ref[...]) + pb_ref[0:1, :]
    tr = t[:N]
    mu = jnp.mean(tr, axis=0, keepdims=True)
    var = jnp.mean((tr - mu) ** 2, axis=0, keepdims=True)
    hn = (t - mu) * lax.rsqrt(var + EPS_BN) * pb_ref[1:2, :] + pb_ref[2:3, :]
    h = jnp.maximum(hn, 0.0)
    hs2 = jnp.dot(h, w_ref[...], preferred_element_type=jnp.float32) * dinv
    rowid = lax.broadcasted_iota(jnp.int32, (NP, H), 0)
    out_ref[...] = jnp.where(rowid < N, hs2, 0.0)


def _tc_final(agg_ref, hs_ref, dinv_ref, pm_ref,
              w1_ref, w2_ref, w3_ref, w4_ref, wo_ref, out_ref):
    dinv = dinv_ref[...][:, 0:1]
    t = dinv * (agg_ref[0] + agg_ref[1] + hs_ref[...]) + pm_ref[0:1, :]
    tr = t[:N]
    mu = jnp.mean(tr, axis=0, keepdims=True)
    var = jnp.mean((tr - mu) ** 2, axis=0, keepdims=True)
    h = jnp.maximum(
        (tr - mu) * lax.rsqrt(var + EPS_BN) * pm_ref[1:2, :] + pm_ref[2:3, :],
        0.0,
    )
    ws = [w1_ref, w2_ref, w3_ref, w4_ref]
    for j in range(4):
        h = jnp.dot(h, ws[j][...], preferred_element_type=jnp.float32)
        h = jnp.maximum(h + pm_ref[3 + 3 * j:4 + 3 * j, :], 0.0)
        mu2 = jnp.mean(h, axis=-1, keepdims=True)
        var2 = jnp.mean((h - mu2) ** 2, axis=-1, keepdims=True)
        h = ((h - mu2) * lax.rsqrt(var2 + EPS_LN) * pm_ref[4 + 3 * j:5 + 3 * j, :]
             + pm_ref[5 + 3 * j:6 + 3 * j, :])
    out = jnp.dot(h, wo_ref[...], preferred_element_type=jnp.float32)
    out_ref[...] = out + pm_ref[15:16, :C]


def _sd(shape):
    return jax.ShapeDtypeStruct(shape, jnp.float32)


def kernel(x, edge_index, params):
    src = edge_index[0]
    dst = edge_index[1]
    pad = jnp.full((EPAD - E,), N, jnp.int32)
    src_p = jnp.concatenate([src, pad]).reshape(NW, CHUNKS, B)
    dst_p = jnp.concatenate([dst, pad]).reshape(NW, CHUNKS, B)
    x_p = jnp.zeros((NP, D), jnp.float32).at[:N].set(x)
    zeros8 = jnp.zeros((NP, 8), jnp.float32)
    zerosH = jnp.zeros((NP, H), jnp.float32)
    ones8 = jnp.ones((B, 8), jnp.float32)

    sc_degree, sc_scatter = _sc_kernels()
    deg_part = sc_degree(dst_p, ones8, zeros8)
    dinv, hs = pl.pallas_call(
        _tc_first, out_shape=[_sd((NP, 8)), _sd((NP, H))]
    )(deg_part, x_p, params['conv_W'][0])

    out = None
    for i in range(3):
        agg = sc_scatter(hs, src_p, dst_p, zerosH)
        if i < 2:
            pb = jnp.concatenate([
                params['conv_b'][i][None, :],
                params['bn_g'][i][None, :],
                params['bn_b'][i][None, :],
                jnp.zeros((5, H), jnp.float32),
            ])
            hs = pl.pallas_call(_tc_mid, out_shape=_sd((NP, H)))(
                agg, hs, dinv, pb, params['conv_W'][i + 1]
            )
        else:
            rows = [params['conv_b'][2][None, :],
                    params['bn_g'][2][None, :],
                    params['bn_b'][2][None, :]]
            for j in range(4):
                rows += [params['head_b'][j][None, :],
                         params['ln_g'][j][None, :],
                         params['ln_b'][j][None, :]]
            rows.append(jnp.pad(params['out_b'], (0, H - C))[None, :])
            pm = jnp.concatenate(rows)
            out = pl.pallas_call(_tc_final, out_shape=_sd((N, C)))(
                agg, hs, dinv, pm,
                params['head_W'][0], params['head_W'][1],
                params['head_W'][2], params['head_W'][3],
                params['out_W'],
            )
    return out


# speedup vs baseline: 41.3553x; 1.6099x over previous
import functools

import jax
import jax.numpy as jnp
from jax import lax
from jax.experimental import pallas as pl
from jax.experimental.pallas import tpu as pltpu
from jax.experimental.pallas import tpu_sc as plsc

N = 10000
E = 640000
D = 128
H = 32
C = 16
EPS_BN = 1e-5
EPS_LN = 1e-5

NC = 2
NS = 16
NW = NC * NS
B = 128
NP = 10016
RING = 4
RPT = NP // NS

CH0S, CH1S = 228, 88
CH0D, CH1D = 208, 112
TOTC = 5216
EPAD = TOTC * B


def _sc_degree(dst_hbm, ones_hbm, zeros_hbm, out_hbm,
               didx_all, ones_v, deg_sh, ssem):
    c = lax.axis_index("c")
    s = lax.axis_index("s")
    wid = c * NS + s

    base_c = jnp.where(c == 0, s * CH0D, NS * CH0D + s * CH1D)
    nblk = jnp.where(c == 0, CH0D // 8, CH1D // 8)
    pltpu.sync_copy(zeros_hbm.at[pl.ds(s * RPT, RPT)],
                    deg_sh.at[pl.ds(s * RPT, RPT)])
    pltpu.sync_copy(dst_hbm.at[pl.ds(base_c, CH0D)], didx_all)
    pltpu.sync_copy(ones_hbm, ones_v)
    plsc.subcore_barrier()

    K = 8

    def body(i, carry):
        for b in range(K):
            pltpu.async_copy(ones_v, deg_sh.at[didx_all.at[K * i + b]],
                             ssem, add=True)
        for b in range(K):
            pltpu.make_async_copy(
                ones_v, deg_sh.at[didx_all.at[K * i + b]], ssem).wait()
        return carry

    lax.fori_loop(0, nblk, body, 0)
    plsc.subcore_barrier()
    pltpu.sync_copy(deg_sh.at[pl.ds(s * RPT, RPT)],
                    out_hbm.at[c, pl.ds(s * RPT, RPT)])


def _sc_scatter(hs_hbm, src_hbm, dst_hbm, zeros_hbm, out_hbm,
                sidx_all, didx_all, rows0, rows1, rows2, rows3,
                agg_sh, g0, g1, g2, g3, s0, s1, s2, s3):
    c = lax.axis_index("c")
    s = lax.axis_index("s")
    wid = c * NS + s
    rows = [rows0, rows1, rows2, rows3]
    gsem = [g0, g1, g2, g3]
    ssem = [s0, s1, s2, s3]

    base_c = jnp.where(c == 0, s * CH0S, NS * CH0S + s * CH1S)
    nblk = jnp.where(c == 0, CH0S // RING, CH1S // RING)
    pltpu.sync_copy(zeros_hbm.at[pl.ds(s * RPT, RPT)],
                    agg_sh.at[pl.ds(s * RPT, RPT)])
    pltpu.sync_copy(src_hbm.at[pl.ds(base_c, CH0S)], sidx_all)
    pltpu.sync_copy(dst_hbm.at[pl.ds(base_c, CH0S)], didx_all)
    plsc.subcore_barrier()

    for b in range(RING):
        pltpu.make_async_copy(hs_hbm.at[sidx_all.at[b]],
                              rows[b], gsem[b]).start()

    def body(i, carry):
        for b in range(RING):
            pltpu.make_async_copy(hs_hbm.at[sidx_all.at[RING * i + b]],
                                  rows[b], gsem[b]).wait()
        for b in range(RING):
            pltpu.async_copy(rows[b], agg_sh.at[didx_all.at[RING * i + b]],
                             ssem[b], add=True)
        for b in range(RING):
            pltpu.make_async_copy(
                rows[b], agg_sh.at[didx_all.at[RING * i + b]], ssem[b]).wait()

        @pl.when(i < nblk - 1)
        def _():
            for b in range(RING):
                pltpu.make_async_copy(
                    hs_hbm.at[sidx_all.at[RING * (i + 1) + b]],
                    rows[b], gsem[b]).start()
        return carry

    lax.fori_loop(0, nblk, body, 0)
    plsc.subcore_barrier()
    pltpu.sync_copy(agg_sh.at[pl.ds(s * RPT, RPT)],
                    out_hbm.at[c, pl.ds(s * RPT, RPT)])


@functools.lru_cache(maxsize=None)
def _sc_kernels():
    mesh = plsc.VectorSubcoreMesh(
        core_axis_name="c", subcore_axis_name="s", num_cores=NC, num_subcores=NS
    )
    cp = pltpu.CompilerParams(use_tc_tiling_on_sc=False)
    degree = pl.kernel(
        _sc_degree,
        out_type=jax.ShapeDtypeStruct((NC, NP, 8), jnp.float32),
        mesh=mesh,
        compiler_params=cp,
        scratch_types=[
            pltpu.VMEM((CH0D, B), jnp.int32),
            pltpu.VMEM((B, 8), jnp.float32),
            pltpu.VMEM_SHARED((NP, 8), jnp.float32),
            pltpu.SemaphoreType.DMA,
        ],
    )
    scatter = pl.kernel(
        _sc_scatter,
        out_type=jax.ShapeDtypeStruct((NC, NP, H), jnp.float32),
        mesh=mesh,
        compiler_params=cp,
        scratch_types=(
            [pltpu.VMEM((CH0S, B), jnp.int32)] * 2
            + [pltpu.VMEM((B, H), jnp.float32)] * RING
            + [pltpu.VMEM_SHARED((NP, H), jnp.float32)]
            + [pltpu.SemaphoreType.DMA] * (2 * RING)
        ),
    )
    return degree, scatter


def _tc_first(deg_ref, x_ref, w_ref, dinv_ref, hs_ref):
    deg = deg_ref[0] + deg_ref[1]
    dinv = lax.rsqrt(deg + 1.0)
    dinv_ref[...] = dinv
    h = jnp.dot(x_ref[...], w_ref[...], preferred_element_type=jnp.float32)
    hs_ref[...] = h * dinv[:, 0:1]


def _tc_mid(agg_ref, hs_ref, dinv_ref, pb_ref, w_ref, out_ref):
    dinv = dinv_ref[...][:, 0:1]
    t = dinv * (agg_ref[0] + agg_ref[1] + hs_ref[...]) + pb_ref[0:1, :]
    tr = t[:N]
    mu = jnp.mean(tr, axis=0, keepdims=True)
    var = jnp.mean((tr - mu) ** 2, axis=0, keepdims=True)
    hn = (t - mu) * lax.rsqrt(var + EPS_BN) * pb_ref[1:2, :] + pb_ref[2:3, :]
    h = jnp.maximum(hn, 0.0)
    hs2 = jnp.dot(h, w_ref[...], preferred_element_type=jnp.float32) * dinv
    rowid = lax.broadcasted_iota(jnp.int32, (NP, H), 0)
    out_ref[...] = jnp.where(rowid < N, hs2, 0.0)


def _tc_final(agg_ref, hs_ref, dinv_ref, pm_ref,
              w1_ref, w2_ref, w3_ref, w4_ref, wo_ref, out_ref):
    dinv = dinv_ref[...][:, 0:1]
    t = dinv * (agg_ref[0] + agg_ref[1] + hs_ref[...]) + pm_ref[0:1, :]
    tr = t[:N]
    mu = jnp.mean(tr, axis=0, keepdims=True)
    var = jnp.mean((tr - mu) ** 2, axis=0, keepdims=True)
    h = jnp.maximum(
        (tr - mu) * lax.rsqrt(var + EPS_BN) * pm_ref[1:2, :] + pm_ref[2:3, :],
        0.0,
    )
    ws = [w1_ref, w2_ref, w3_ref, w4_ref]
    for j in range(4):
        h = jnp.dot(h, ws[j][...], preferred_element_type=jnp.float32)
        h = jnp.maximum(h + pm_ref[3 + 3 * j:4 + 3 * j, :], 0.0)
        mu2 = jnp.mean(h, axis=-1, keepdims=True)
        var2 = jnp.mean((h - mu2) ** 2, axis=-1, keepdims=True)
        h = ((h - mu2) * lax.rsqrt(var2 + EPS_LN) * pm_ref[4 + 3 * j:5 + 3 * j, :]
             + pm_ref[5 + 3 * j:6 + 3 * j, :])
    out = jnp.dot(h, wo_ref[...], preferred_element_type=jnp.float32)
    out_ref[...] = out + pm_ref[15:16, :C]


def _sd(shape):
    return jax.ShapeDtypeStruct(shape, jnp.float32)


def kernel(x, edge_index, params):
    src = edge_index[0]
    dst = edge_index[1]
    pad = jnp.full((EPAD - E,), N, jnp.int32)
    src_p = jnp.concatenate([src, pad]).reshape(TOTC, B)
    dst_p = jnp.concatenate([dst, pad]).reshape(TOTC, B)
    x_p = jnp.zeros((NP, D), jnp.float32).at[:N].set(x)
    zeros8 = jnp.zeros((NP, 8), jnp.float32)
    zerosH = jnp.zeros((NP, H), jnp.float32)
    ones8 = jnp.ones((B, 8), jnp.float32)

    sc_degree, sc_scatter = _sc_kernels()
    deg_part = sc_degree(dst_p, ones8, zeros8)
    dinv, hs = pl.pallas_call(
        _tc_first, out_shape=[_sd((NP, 8)), _sd((NP, H))]
    )(deg_part, x_p, params['conv_W'][0])

    out = None
    for i in range(3):
        agg = sc_scatter(hs, src_p, dst_p, zerosH)
        if i < 2:
            pb = jnp.concatenate([
                params['conv_b'][i][None, :],
                params['bn_g'][i][None, :],
                params['bn_b'][i][None, :],
                jnp.zeros((5, H), jnp.float32),
            ])
            hs = pl.pallas_call(_tc_mid, out_shape=_sd((NP, H)))(
                agg, hs, dinv, pb, params['conv_W'][i + 1]
            )
        else:
            rows = [params['conv_b'][2][None, :],
                    params['bn_g'][2][None, :],
                    params['bn_b'][2][None, :]]
            for j in range(4):
                rows += [params['head_b'][j][None, :],
                         params['ln_g'][j][None, :],
                         params['ln_b'][j][None, :]]
            rows.append(jnp.pad(params['out_b'], (0, H - C))[None, :])
            pm = jnp.concatenate(rows)
            out = pl.pallas_call(_tc_final, out_shape=_sd((N, C)))(
                agg, hs, dinv, pm,
                params['head_W'][0], params['head_W'][1],
                params['head_W'][2], params['head_W'][3],
                params['out_W'],
            )
    return out
